# baseline (speedup 1.0000x reference)
import functools

import jax
import jax.numpy as jnp
from jax import lax
from jax.experimental import pallas as pl
from jax.experimental.pallas import tpu as pltpu
from jax.experimental.pallas import tpu_sc as plsc

_CCH = 256
_L = 16
_TS = 128

_PARAMS = dict(use_tc_tiling_on_sc=True, needs_layout_passes=False)


def _sc_gather_c(N, D, NC, NS):
    NW = NC * NS
    bc = N // NW
    n_ch = bc // _CCH
    assert bc % _CCH == 0 and n_ch % 2 == 0 and n_ch >= 4
    mesh = plsc.VectorSubcoreMesh(core_axis_name="c", subcore_axis_name="s")

    @functools.partial(
        pl.kernel,
        out_type=jax.ShapeDtypeStruct((N, D), jnp.float32),
        mesh=mesh,
        compiler_params=pltpu.CompilerParams(**_PARAMS),
        scratch_types=[
            pltpu.VMEM((bc,), jnp.int32),
            pltpu.VMEM((2, _CCH, D), jnp.float32),
            pltpu.SemaphoreType.DMA,
            pltpu.SemaphoreType.DMA,
        ],
    )
    def body(cidx_hbm, ctab, c_out, cidx_v, rows_v, sem0, sem1):
        wid = lax.axis_index("s") * NC + lax.axis_index("c")
        pltpu.sync_copy(cidx_hbm.at[wid], cidx_v)
        c_base = wid * bc
        sems = (sem0, sem1)

        def issue(chunk, buf):
            def blk(j, carry):
                v16 = cidx_v[pl.ds(chunk * _CCH + j * _L, _L)]
                for k in range(_L):
                    pltpu.async_copy(
                        ctab.at[pl.ds(v16[k], 1)],
                        rows_v.at[buf, pl.ds(j * _L + k, 1)],
                        sems[buf],
                    )
                return carry

            lax.fori_loop(0, _CCH // _L, blk, 0)

        def drain_write(chunk, buf):
            pltpu.make_async_copy(
                c_out.at[pl.ds(0, _CCH)], rows_v.at[buf], sems[buf]
            ).wait()
            pltpu.sync_copy(
                rows_v.at[buf], c_out.at[pl.ds(c_base + chunk * _CCH, _CCH)]
            )

        issue(0, 0)
        issue(1, 1)

        def step(p, carry):
            drain_write(2 * p, 0)
            issue(2 * p + 2, 0)
            drain_write(2 * p + 1, 1)
            issue(2 * p + 3, 1)
            return carry

        lax.fori_loop(0, n_ch // 2 - 1, step, 0)
        drain_write(n_ch - 2, 0)
        drain_write(n_ch - 1, 1)

    return body


def _sc_gather_w(B, D, V, NC, NS):
    NW = NC * NS
    bw = B // NW
    assert bw % _L == 0
    mesh = plsc.VectorSubcoreMesh(core_axis_name="c", subcore_axis_name="s")

    @functools.partial(
        pl.kernel,
        out_type=jax.ShapeDtypeStruct((B, D), jnp.float32),
        mesh=mesh,
        compiler_params=pltpu.CompilerParams(**_PARAMS),
        scratch_types=[
            pltpu.VMEM((bw,), jnp.int32),
            pltpu.VMEM((bw, D), jnp.float32),
            pltpu.SemaphoreType.DMA,
        ],
    )
    def body(widx_hbm, wtab, w_out, widx_v, rows_v, sem):
        wid = lax.axis_index("s") * NC + lax.axis_index("c")
        pltpu.sync_copy(widx_hbm.at[wid], widx_v)

        def issue(j, carry):
            v16 = widx_v[pl.ds(j * _L, _L)]
            for k in range(_L):
                pltpu.async_copy(
                    wtab.at[pl.ds(v16[k], 1)],
                    rows_v.at[pl.ds(j * _L + k, 1)],
                    sem,
                )
            return carry

        lax.fori_loop(0, bw // _L, issue, 0)
        pltpu.make_async_copy(w_out.at[pl.ds(0, bw)], rows_v, sem).wait()
        pltpu.sync_copy(rows_v, w_out.at[pl.ds(wid * bw, bw)])

    return body


def kernel(words, contexts, w_embedding, c_embedding):
    (B,) = words.shape
    _, CTX = contexts.shape
    V, D = w_embedding.shape
    N = B * CTX
    info = plsc.get_sparse_core_info()
    NC, NS = info.num_cores, info.num_subcores
    NW = NC * NS

    w_idx = words.reshape(NW, B // NW)
    c_idx = contexts.T.reshape(NW, N // NW)
    w_out = _sc_gather_w(B, D, V, NC, NS)(w_idx, w_embedding)
    c_out = _sc_gather_c(N, D, NC, NS)(c_idx, c_embedding)
    return w_out, c_out.reshape(CTX, B, D).transpose(1, 0, 2)
